# baseline (speedup 1.0000x reference)
import functools

import jax
import jax.numpy as jnp
from jax import lax
from jax.experimental import pallas as pl
from jax.experimental.pallas import tpu as pltpu
from jax.experimental.pallas import tpu_sc as plsc

_D = 64
_NC = 2
_NS = 16
_NW = _NC * _NS
_BB = 128
_DB = _D // 8


@functools.cache
def _gather_call(seq: int, nbb: int):
    n_units = seq * nbb
    u_per_w = n_units // _NW
    n_super = u_per_w // 4
    mesh = plsc.VectorSubcoreMesh(core_axis_name="c", subcore_axis_name="s")

    @functools.partial(
        pl.kernel,
        out_type=jax.ShapeDtypeStruct((seq, _DB, nbb, 8, _BB), jnp.float32),
        mesh=mesh,
        scratch_types=[
            pltpu.VMEM((u_per_w, _BB), jnp.int32),
            pltpu.VMEM((_BB, _D), jnp.float32),
            pltpu.VMEM((_BB, _D), jnp.float32),
            pltpu.VMEM((_BB, _D), jnp.float32),
            pltpu.VMEM((_BB, _D), jnp.float32),
            pltpu.VMEM((_BB, _D + 1), jnp.float32),
            pltpu.VMEM((_DB, 8, _BB), jnp.float32),
            pltpu.VMEM((_DB, 8, _BB), jnp.float32),
            pltpu.SemaphoreType.DMA,
            pltpu.SemaphoreType.DMA,
            pltpu.SemaphoreType.DMA,
            pltpu.SemaphoreType.DMA,
            pltpu.SemaphoreType.DMA,
            pltpu.SemaphoreType.DMA,
        ],
        compiler_params=pltpu.CompilerParams(
            use_tc_tiling_on_sc=False, needs_layout_passes=False),
    )
    def body(idx_hbm, table_hbm, out_hbm, idx_all, rows0, rows1, rows2, rows3,
             rskew, t0, t1, gsem0, gsem1, gsem2, gsem3, wsem0, wsem1):
        wid = lax.axis_index("s") * _NC + lax.axis_index("c")
        u0 = wid * u_per_w
        rows = (rows0, rows1, rows2, rows3)
        ts = (t0, t1)
        gsems = (gsem0, gsem1, gsem2, gsem3)
        wsems = (wsem0, wsem1)
        lane = lax.iota(jnp.int32, 16)

        pltpu.sync_copy(idx_hbm.at[pl.ds(u0, u_per_w)], idx_all)

        def fire_gather(j, b):
            pltpu.async_copy(table_hbm.at[idx_all.at[j]], rows[b], gsems[b])

        def wait_gather(b):
            pltpu.make_async_copy(
                table_hbm.at[pl.ds(0, _BB)], rows[b], gsems[b]).wait()

        def transpose(rb, tb):
            src = rows[rb]
            dst = ts[tb]

            def skew_body(brow, carry):
                for k in range(_D // 16):
                    rskew[brow, pl.ds(16 * k, 16)] = src[brow, pl.ds(16 * k, 16)]
                return carry

            lax.fori_loop(0, _BB, skew_body, 0)

            def b0_body(b0, carry):
                rv = lane + b0
                for db in range(_DB):
                    for di in range(8):
                        col = jnp.full((16,), 8 * db + di, jnp.int32)
                        v = plsc.load_gather(rskew, [rv, col])
                        dst[db, di, pl.ds(b0, 16)] = v
                return carry

            lax.fori_loop(0, _BB // 16, lambda g, c: b0_body(g * 16, c), 0)

        def fire_write(j, b):
            u = u0 + j
            s = u // nbb
            bb = u % nbb
            pltpu.async_copy(ts[b], out_hbm.at[s, :, bb], wsems[b])

        def wait_write(b):
            pltpu.make_async_copy(ts[b], out_hbm.at[0, :, 0], wsems[b]).wait()

        for r in range(4):
            fire_gather(r, r)

        def super_body(sidx, carry):
            for r in range(4):
                j = 4 * sidx + r
                wait_gather(r)
                if r < 2:
                    @pl.when(sidx > 0)
                    def _():
                        wait_write(r % 2)
                else:
                    wait_write(r % 2)
                transpose(r, r % 2)

                @pl.when(sidx < n_super - 1)
                def _():
                    fire_gather(j + 4, r)
                fire_write(j, r % 2)
            return carry

        lax.fori_loop(0, n_super, super_body, 0)
        wait_write(0)
        wait_write(1)

    return body


def kernel(input_ids, embed_table):
    batch, seq = input_ids.shape
    vocab = embed_table.shape[0]
    nbb = batch // _BB
    idx_t = input_ids.T.astype(jnp.int32).reshape(seq * nbb, _BB)
    tab_lin = jax.lax.optimization_barrier(embed_table.reshape(vocab // 2, 2 * _D))
    tab2 = tab_lin.reshape(vocab, _D)
    out5 = _gather_call(seq, nbb)(idx_t, tab2)
    return out5.transpose(2, 4, 0, 1, 3).reshape(batch, seq, _D)


# speedup vs baseline: 1.0183x; 1.0183x over previous
import functools

import jax
import jax.numpy as jnp
from jax import lax
from jax.experimental import pallas as pl
from jax.experimental.pallas import tpu as pltpu
from jax.experimental.pallas import tpu_sc as plsc

_D = 64
_NC = 2
_NS = 16
_NW = _NC * _NS
_BB = 128
_DB = _D // 8


@functools.cache
def _gather_call(seq: int, nbb: int):
    n_units = seq * nbb
    u_per_w = n_units // _NW
    n_super = u_per_w // 4
    mesh = plsc.VectorSubcoreMesh(core_axis_name="c", subcore_axis_name="s")

    @functools.partial(
        pl.kernel,
        out_type=jax.ShapeDtypeStruct((seq, _DB, nbb, 8, _BB), jnp.float32),
        mesh=mesh,
        scratch_types=[
            pltpu.VMEM((u_per_w, _BB), jnp.int32),
            pltpu.VMEM((_BB, _D), jnp.float32),
            pltpu.VMEM((_BB, _D), jnp.float32),
            pltpu.VMEM((_BB, _D), jnp.float32),
            pltpu.VMEM((_BB, _D), jnp.float32),
            pltpu.VMEM((_BB, _D + 1), jnp.float32),
            pltpu.VMEM((_DB, 8, _BB), jnp.float32),
            pltpu.VMEM((_DB, 8, _BB), jnp.float32),
            pltpu.SemaphoreType.DMA,
            pltpu.SemaphoreType.DMA,
            pltpu.SemaphoreType.DMA,
            pltpu.SemaphoreType.DMA,
            pltpu.SemaphoreType.DMA,
            pltpu.SemaphoreType.DMA,
        ],
        compiler_params=pltpu.CompilerParams(
            use_tc_tiling_on_sc=False, needs_layout_passes=False),
    )
    def body(idx_hbm, table_hbm, out_hbm, idx_all, rows0, rows1, rows2, rows3,
             rskew, t0, t1, gsem0, gsem1, gsem2, gsem3, wsem0, wsem1):
        wid = lax.axis_index("s") * _NC + lax.axis_index("c")
        u0 = wid * u_per_w
        rows = (rows0, rows1, rows2, rows3)
        ts = (t0, t1)
        gsems = (gsem0, gsem1, gsem2, gsem3)
        wsems = (wsem0, wsem1)
        lane = lax.iota(jnp.int32, 16)

        pltpu.sync_copy(idx_hbm.at[pl.ds(u0, u_per_w)], idx_all)

        def fire_gather(j, b):
            pltpu.async_copy(table_hbm.at[idx_all.at[j]], rows[b], gsems[b])

        def wait_gather(b):
            pltpu.make_async_copy(
                table_hbm.at[pl.ds(0, _BB)], rows[b], gsems[b]).wait()

        cols = [jnp.full((16,), d, jnp.int32) for d in range(_D)]

        def transpose(rb, tb):
            src = rows[rb]
            dst = ts[tb]

            def skew_body(b8, carry):
                for bi in range(8):
                    brow = b8 * 8 + bi
                    for k in range(_D // 16):
                        rskew[brow, pl.ds(16 * k, 16)] = (
                            src[brow, pl.ds(16 * k, 16)])
                return carry

            lax.fori_loop(0, _BB // 8, skew_body, 0)

            def b0_body(g, carry):
                rv = lane + g * 16
                for db in range(_DB):
                    for di in range(8):
                        v = plsc.load_gather(rskew, [rv, cols[8 * db + di]])
                        dst[db, di, pl.ds(g * 16, 16)] = v
                return carry

            lax.fori_loop(0, _BB // 16, b0_body, 0)

        def fire_write(j, b):
            u = u0 + j
            s = u // nbb
            bb = u % nbb
            pltpu.async_copy(ts[b], out_hbm.at[s, :, bb], wsems[b])

        def wait_write(b):
            pltpu.make_async_copy(ts[b], out_hbm.at[0, :, 0], wsems[b]).wait()

        for r in range(4):
            fire_gather(r, r)

        def super_body(sidx, carry):
            for r in range(4):
                j = 4 * sidx + r
                wait_gather(r)
                if r < 2:
                    @pl.when(sidx > 0)
                    def _():
                        wait_write(r % 2)
                else:
                    wait_write(r % 2)
                transpose(r, r % 2)

                @pl.when(sidx < n_super - 1)
                def _():
                    fire_gather(j + 4, r)
                fire_write(j, r % 2)
            return carry

        lax.fori_loop(0, n_super, super_body, 0)
        wait_write(0)
        wait_write(1)

    return body


def kernel(input_ids, embed_table):
    batch, seq = input_ids.shape
    vocab = embed_table.shape[0]
    nbb = batch // _BB
    idx_t = input_ids.T.astype(jnp.int32).reshape(seq * nbb, _BB)
    tab_lin = jax.lax.optimization_barrier(embed_table.reshape(vocab // 2, 2 * _D))
    tab2 = tab_lin.reshape(vocab, _D)
    out5 = _gather_call(seq, nbb)(idx_t, tab2)
    return out5.transpose(2, 4, 0, 1, 3).reshape(batch, seq, _D)
